# baseline (speedup 1.0000x reference)
import functools

import jax
import jax.numpy as jnp
import numpy as np
from jax import lax
from jax.experimental import pallas as pl
from jax.experimental.pallas import tpu as pltpu
from jax.experimental.pallas import tpu_sc as plsc

B = 16384
F = 26
V = 100000
D = 16

NC = 2
NS = 16
L = 16
NW = NC * NS
BPW = B // NW
C = 128
NCHUNK = BPW // C
N = C * F
GW = 128
G = N // GW


def _fm_body(xi_hbm, xv_hbm, w1_hbm, w2_hbm, out_hbm,
             idxt_v, idxf_v, xvt_v, w1_v, rows_v, ob_v, sem):
    wid = lax.axis_index("s") * NC + lax.axis_index("c")
    base = wid * BPW
    lanes = lax.iota(jnp.int32, L)
    _dn = lax.GatherDimensionNumbers(
        offset_dims=(), collapsed_slice_dims=(0,), start_index_map=(0,))

    def _shuf(x, perm):
        return lax.gather(x, perm[:, None], dimension_numbers=_dn,
                          slice_sizes=(1,),
                          mode=lax.GatherScatterMode.PROMISE_IN_BOUNDS)

    def _lane_sum(x):
        for k in (8, 4, 2, 1):
            x = x + _shuf(x, lanes ^ k)
        return x

    def chunk_body(k, _):
        b0 = pl.multiple_of(base + k * C, 128)

        pltpu.sync_copy(xi_hbm.at[:, pl.ds(b0, C)], idxt_v)
        pltpu.sync_copy(xv_hbm.at[:, pl.ds(b0, C)], xvt_v)

        for f in range(F):
            for c in range(C // L):
                j = f * C + c * L
                vec = idxt_v[f, pl.ds(c * L, L)] + jnp.int32(f * V)
                idxf_v[j // GW, pl.ds(j % GW, L)] = vec

        copies = []
        for g in range(G):
            copies.append(pltpu.async_copy(
                w2_hbm.at[idxf_v.at[g]],
                rows_v.at[pl.ds(g * GW, GW)], sem))
            copies.append(pltpu.async_copy(
                w1_hbm.at[idxf_v.at[g]],
                w1_v.at[pl.ds(g * GW, GW)], sem))
        for cp in copies:
            cp.wait()

        def bbody(bg, _):
            facc = jnp.zeros((L,), jnp.float32)
            xvl = []
            for f in range(F):
                xvrow = xvt_v[f, pl.ds(bg * L, L)]
                w1row = w1_v[pl.ds(f * C + bg * L, L)]
                facc = facc + w1row * xvrow
                xvl.append(xvrow)
            ovec = facc
            for i in range(L):
                b = bg * L + i
                acc = jnp.zeros((L,), jnp.float32)
                acc2 = jnp.zeros((L,), jnp.float32)
                for f in range(F):
                    t = rows_v[f * C + b, :] * xvl[f][i]
                    acc = acc + t
                    acc2 = acc2 + t * t
                s = _lane_sum(acc * acc - acc2)
                ovec = jnp.where(lanes == i, ovec + jnp.float32(0.5) * s, ovec)
            ob_v[pl.ds(bg * L, L)] = ovec
            return 0

        lax.fori_loop(0, C // L, bbody, 0)
        pltpu.sync_copy(ob_v, out_hbm.at[pl.ds(b0, C)])
        return 0

    lax.fori_loop(0, NCHUNK, chunk_body, 0)


@jax.jit
def _fm_kernel(xi_t, xv_t, w1_flat, w2_2d):
    mesh = plsc.VectorSubcoreMesh(core_axis_name="c", subcore_axis_name="s")
    run = pl.kernel(
        _fm_body,
        mesh=mesh,
        compiler_params=pltpu.CompilerParams(use_tc_tiling_on_sc=False),
        out_type=jax.ShapeDtypeStruct((B,), jnp.float32),
        scratch_types=[
            pltpu.VMEM((F, C), jnp.int32),
            pltpu.VMEM((G, GW), jnp.int32),
            pltpu.VMEM((F, C), jnp.float32),
            pltpu.VMEM((N,), jnp.float32),
            pltpu.VMEM((N, D), jnp.float32),
            pltpu.VMEM((C,), jnp.float32),
            pltpu.SemaphoreType.DMA,
        ],
    )
    return run(xi_t, xv_t, w1_flat, w2_2d)


def kernel(Xi, Xv, W1, W2, bias):
    xi_t = Xi[:, :, 0].T.astype(jnp.int32)
    xv_t = Xv.T
    w1_flat = W1.reshape(F * V)
    w2_2d = W2.reshape(F * V, D)
    return _fm_kernel(xi_t, xv_t, w1_flat, w2_2d) + bias
